# baseline (speedup 1.0000x reference)
import jax
import jax.numpy as jnp
from jax import lax
from jax.experimental import pallas as pl
from jax.experimental.pallas import tpu as pltpu, tpu_sc as plsc

D = 64
SCALE = 8.0

_info = plsc.get_sparse_core_info()
NC, NS, L = _info.num_cores, _info.num_subcores, _info.num_lanes
NW = NC * NS

B = 16384 * 50
B_PER_W = B // NW
CB = 320
NCHUNK = B_PER_W // CB
NBUF = 4
OBUF = 2
NROUND = NCHUNK // NBUF


def _body(table_hbm, idx_hbm, out_hbm, *s):
    idx_v = s[0:NBUF]
    rows_v = s[NBUF:2 * NBUF]
    obuf_v = s[2 * NBUF:2 * NBUF + OBUF]
    gsem = s[2 * NBUF + OBUF:3 * NBUF + OBUF]
    osem = s[3 * NBUF + OBUF:3 * NBUF + OBUF + OBUF]

    wid = lax.axis_index("s") * NC + lax.axis_index("c")
    base = wid * B_PER_W

    def idx_and_gather(b, c):
        pltpu.sync_copy(idx_hbm.at[pl.ds(base + c * CB, CB)], idx_v[b])
        pltpu.async_copy(table_hbm.at[idx_v[b]], rows_v[b], gsem[b])

    def wait_gather(b):
        pltpu.make_async_copy(table_hbm.at[idx_v[b]], rows_v[b], gsem[b]).wait()

    def scale(b, ob):
        def row(i, _):
            for j in range(D // L):
                obuf_v[ob][pl.ds(i * D + j * L, L)] = (
                    rows_v[b][i, pl.ds(j * L, L)] * SCALE)
            return 0
        lax.fori_loop(0, CB, row, 0, unroll=4)

    def start_out(ob, c):
        pltpu.async_copy(
            obuf_v[ob], out_hbm.at[pl.ds((base + c * CB) * D, CB * D)],
            osem[ob])

    def wait_out(ob):
        pltpu.make_async_copy(
            obuf_v[ob], out_hbm.at[pl.ds(0, CB * D)], osem[ob]).wait()

    for b in range(NBUF):
        idx_and_gather(b, b)

    def process(c, b, first, prefetch):
        ob = b % OBUF
        wait_gather(b)
        if not first:
            wait_out(ob)
        scale(b, ob)
        start_out(ob, c)
        if prefetch:
            idx_and_gather(b, c + NBUF)

    for b in range(NBUF):
        process(b, b, first=(b < OBUF), prefetch=True)

    def rnd(g, _):
        for b in range(NBUF):
            process(g * NBUF + b, b, first=False, prefetch=True)
        return 0
    lax.fori_loop(1, NROUND - 1, rnd, 0)

    for b in range(NBUF):
        process((NROUND - 1) * NBUF + b, b, first=False, prefetch=False)

    for ob in range(OBUF):
        wait_out(ob)


@jax.jit
def _embed(table, idx):
    mesh = plsc.VectorSubcoreMesh(core_axis_name="c", subcore_axis_name="s")
    f = pl.kernel(
        _body,
        out_type=jax.ShapeDtypeStruct((B * D,), jnp.float32),
        mesh=mesh,
        scratch_types=(
            [pltpu.VMEM((CB,), jnp.int32) for _ in range(NBUF)]
            + [pltpu.VMEM((CB, D), jnp.float32) for _ in range(NBUF)]
            + [pltpu.VMEM((CB * D,), jnp.float32) for _ in range(OBUF)]
            + [pltpu.SemaphoreType.DMA for _ in range(NBUF + OBUF)]
        ),
        compiler_params=pltpu.CompilerParams(use_tc_tiling_on_sc=False),
    )
    return f(table, idx)


def kernel(x, table):
    idx = x.reshape(-1).astype(jnp.int32)
    out = _embed(table, idx)
    return out.reshape(x.shape[0], x.shape[1], D)
